# Initial kernel scaffold; baseline (speedup 1.0000x reference)
import jax
import jax.numpy as jnp
from jax.experimental import pallas as pl


def kernel(x, edge_index, batch, W1, b1, W2, b2, W3, b3):
    raise NotImplementedError("write your pallas kernel here")


# speedup vs baseline: 10.9112x; 10.9112x over previous
import functools

import jax
import jax.numpy as jnp
from jax import lax
from jax.experimental import pallas as pl
from jax.experimental.pallas import tpu as pltpu
from jax.experimental.pallas import tpu_sc as plsc

N = 10000
E = 320000
D = 128
G = 128

NC = 2
NS = 16
NW = NC * NS
EP = E // NW
CH = 80
NCH = EP // CH
RT = 640
NP = 10240
DW = 128

_mesh = plsc.VectorSubcoreMesh(core_axis_name="c", subcore_axis_name="s")


@functools.partial(
    pl.kernel,
    out_type=jax.ShapeDtypeStruct((2 * NP, DW), jnp.float32),
    mesh=_mesh,
    scratch_types=[
        pltpu.VMEM((CH,), jnp.int32),
        pltpu.VMEM((CH, DW), jnp.float32),
        pltpu.VMEM_SHARED((NP, DW), jnp.float32),
        pltpu.SemaphoreType.DMA,
    ],
)
def _sc_degree(dst_hbm, ones_hbm, zeros_hbm, out_hbm, idx_d, ones_v, acc, sem):
    c = lax.axis_index("c")
    s = lax.axis_index("s")
    wid = s * NC + c
    stripe = NP // NS
    pltpu.sync_copy(zeros_hbm.at[pl.ds(s * stripe, stripe)],
                    acc.at[pl.ds(s * stripe, stripe)])
    pltpu.sync_copy(ones_hbm, ones_v)
    plsc.subcore_barrier()

    def body(j, carry):
        base = wid * EP + j * CH
        pltpu.sync_copy(dst_hbm.at[pl.ds(base, CH)], idx_d)
        pltpu.sync_copy(ones_v, acc.at[idx_d], add=True)
        return carry

    lax.fori_loop(0, NCH, body, 0)
    plsc.subcore_barrier()
    pltpu.sync_copy(acc.at[pl.ds(s * stripe, stripe)],
                    out_hbm.at[pl.ds(c * NP + s * stripe, stripe)])


@functools.partial(
    pl.kernel,
    out_type=jax.ShapeDtypeStruct((2 * NP, D), jnp.float32),
    mesh=_mesh,
    scratch_types=[
        pltpu.VMEM((CH,), jnp.int32),
        pltpu.VMEM((CH,), jnp.int32),
        pltpu.VMEM((CH, D), jnp.float32),
        pltpu.VMEM_SHARED((NP, D), jnp.float32),
        pltpu.SemaphoreType.DMA,
    ],
)
def _sc_scatter(src_hbm, dst_hbm, g_hbm, zeros_hbm, out_hbm,
                idx_s, idx_d, rows, acc, sem):
    c = lax.axis_index("c")
    s = lax.axis_index("s")
    wid = s * NC + c
    pltpu.sync_copy(zeros_hbm.at[pl.ds(s * RT, RT)], acc.at[pl.ds(s * RT, RT)])
    plsc.subcore_barrier()

    def body(j, carry):
        base = wid * EP + j * CH
        pltpu.sync_copy(src_hbm.at[pl.ds(base, CH)], idx_s)
        pltpu.async_copy(g_hbm.at[idx_s], rows, sem).wait()
        pltpu.sync_copy(dst_hbm.at[pl.ds(base, CH)], idx_d)
        pltpu.sync_copy(rows, acc.at[idx_d], add=True)
        return carry

    lax.fori_loop(0, NCH, body, 0)
    plsc.subcore_barrier()
    pltpu.sync_copy(acc.at[pl.ds(s * RT, RT)],
                    out_hbm.at[pl.ds(c * NP + s * RT, RT)])


BLK = 2048
NBLK = NP // BLK


def _tc_first_body(deg2_ref, x_ref, w_ref, g_ref):
    dinv = lax.rsqrt(deg2_ref[0] + deg2_ref[1] + 1.0)
    g_ref[...] = jnp.dot(x_ref[...], w_ref[...],
                         preferred_element_type=jnp.float32) * dinv


def _tc_mid_body(deg2_ref, a_ref, g_ref, b_ref, w_ref, out_ref):
    dinv = lax.rsqrt(deg2_ref[0] + deg2_ref[1] + 1.0)
    h = (a_ref[0] + a_ref[1] + g_ref[...]) * dinv + b_ref[...]
    h = jnp.maximum(h, 0.0)
    out_ref[...] = jnp.dot(h, w_ref[...],
                           preferred_element_type=jnp.float32) * dinv


def _tc_pool_body(deg2_ref, a_ref, g_ref, b_ref, batch_ref, out_ref, cnt_ref):
    i = pl.program_id(0)
    dinv = lax.rsqrt(deg2_ref[0] + deg2_ref[1] + 1.0)
    h = (a_ref[0] + a_ref[1] + g_ref[...]) * dinv + b_ref[...]
    bt = batch_ref[...]
    gid = lax.broadcasted_iota(jnp.int32, (G, BLK), 0)
    onehot = (gid == bt).astype(jnp.float32)

    @pl.when(i == 0)
    def _():
        out_ref[...] = jnp.zeros_like(out_ref)
        cnt_ref[...] = jnp.zeros_like(cnt_ref)

    out_ref[...] += jnp.dot(onehot, h, preferred_element_type=jnp.float32)
    cnt_ref[...] += jnp.sum(onehot, axis=1, keepdims=True)

    @pl.when(i == NBLK - 1)
    def _():
        out_ref[...] = out_ref[...] / jnp.maximum(cnt_ref[...], 1.0)


def _tc_first(deg2, x, w):
    return pl.pallas_call(
        _tc_first_body,
        grid=(NBLK,),
        in_specs=[
            pl.BlockSpec((2, BLK, 1), lambda i: (0, i, 0)),
            pl.BlockSpec((BLK, D), lambda i: (i, 0)),
            pl.BlockSpec((D, D), lambda i: (0, 0)),
        ],
        out_specs=pl.BlockSpec((BLK, D), lambda i: (i, 0)),
        out_shape=jax.ShapeDtypeStruct((NP, D), jnp.float32),
    )(deg2, x, w)


def _tc_mid(deg2, a, g, b, w):
    return pl.pallas_call(
        _tc_mid_body,
        grid=(NBLK,),
        in_specs=[
            pl.BlockSpec((2, BLK, 1), lambda i: (0, i, 0)),
            pl.BlockSpec((2, BLK, D), lambda i: (0, i, 0)),
            pl.BlockSpec((BLK, D), lambda i: (i, 0)),
            pl.BlockSpec((1, D), lambda i: (0, 0)),
            pl.BlockSpec((D, D), lambda i: (0, 0)),
        ],
        out_specs=pl.BlockSpec((BLK, D), lambda i: (i, 0)),
        out_shape=jax.ShapeDtypeStruct((NP, D), jnp.float32),
    )(deg2, a, g, b, w)


def _tc_pool(deg2, a, g, b, batch_row):
    out, _ = pl.pallas_call(
        _tc_pool_body,
        grid=(NBLK,),
        in_specs=[
            pl.BlockSpec((2, BLK, 1), lambda i: (0, i, 0)),
            pl.BlockSpec((2, BLK, D), lambda i: (0, i, 0)),
            pl.BlockSpec((BLK, D), lambda i: (i, 0)),
            pl.BlockSpec((1, D), lambda i: (0, 0)),
            pl.BlockSpec((1, BLK), lambda i: (0, i)),
        ],
        out_specs=[
            pl.BlockSpec((G, D), lambda i: (0, 0)),
            pl.BlockSpec((G, 1), lambda i: (0, 0)),
        ],
        out_shape=[
            jax.ShapeDtypeStruct((G, D), jnp.float32),
            jax.ShapeDtypeStruct((G, 1), jnp.float32),
        ],
    )(deg2, a, g, b, batch_row)
    return out


def kernel(x, edge_index, batch, W1, b1, W2, b2, W3, b3):
    x = x.astype(jnp.float32)
    src = edge_index[0]
    dst = edge_index[1]

    ones_rows = jnp.zeros((CH, DW), jnp.float32).at[:, 0].set(1.0)
    zeros_deg = jnp.zeros((NP, DW), jnp.float32)
    zeros_nd = jnp.zeros((NP, D), jnp.float32)

    deg_out = _sc_degree(dst, ones_rows, zeros_deg)
    deg2 = deg_out.reshape(2, NP, DW)[:, :, 0:1]

    b1r = b1.reshape(1, D)
    b2r = b2.reshape(1, D)
    b3r = b3.reshape(1, D)
    batch_row = jnp.pad(batch, (0, NP - N), constant_values=-1).reshape(1, NP)
    xp = jnp.pad(x, ((0, NP - N), (0, 0)))

    g1 = _tc_first(deg2, xp, W1)
    a1 = _sc_scatter(src, dst, g1, zeros_nd).reshape(2, NP, D)
    g2 = _tc_mid(deg2, a1, g1, b1r, W2)
    a2 = _sc_scatter(src, dst, g2, zeros_nd).reshape(2, NP, D)
    g3 = _tc_mid(deg2, a2, g2, b2r, W3)
    a3 = _sc_scatter(src, dst, g3, zeros_nd).reshape(2, NP, D)
    return _tc_pool(deg2, a3, g3, b3r, batch_row)


# speedup vs baseline: 24.2791x; 2.2252x over previous
import functools

import jax
import jax.numpy as jnp
from jax import lax
from jax.experimental import pallas as pl
from jax.experimental.pallas import tpu as pltpu
from jax.experimental.pallas import tpu_sc as plsc

N = 10000
E = 320000
D = 128
G = 128

NC = 2
NS = 16
NW = NC * NS
EP = E // NW
CH = 80
NCH = EP // CH
RT = 640
NP = 10240
DW = 128

_mesh = plsc.VectorSubcoreMesh(core_axis_name="c", subcore_axis_name="s")


@functools.partial(
    pl.kernel,
    out_type=jax.ShapeDtypeStruct((2 * NP, DW), jnp.float32),
    mesh=_mesh,
    scratch_types=[
        pltpu.VMEM((EP,), jnp.int32),
        pltpu.VMEM((CH, DW), jnp.float32),
        pltpu.VMEM_SHARED((NP, DW), jnp.float32),
        pltpu.SemaphoreType.DMA,
    ],
)
def _sc_degree(dst_hbm, ones_hbm, zeros_hbm, out_hbm, idx_d, ones_v, acc, sem):
    c = lax.axis_index("c")
    s = lax.axis_index("s")
    wid = s * NC + c
    stripe = NP // NS
    pltpu.sync_copy(zeros_hbm.at[pl.ds(s * stripe, stripe)],
                    acc.at[pl.ds(s * stripe, stripe)])
    pltpu.sync_copy(ones_hbm, ones_v)
    pltpu.sync_copy(dst_hbm.at[pl.ds(wid * EP, EP)], idx_d)
    plsc.subcore_barrier()

    def body(j, carry):
        pltpu.sync_copy(ones_v, acc.at[idx_d.at[pl.ds(j * CH, CH)]], add=True)
        return carry

    lax.fori_loop(0, NCH, body, 0)
    plsc.subcore_barrier()
    pltpu.sync_copy(acc.at[pl.ds(s * stripe, stripe)],
                    out_hbm.at[pl.ds(c * NP + s * stripe, stripe)])


@functools.partial(
    pl.kernel,
    out_type=jax.ShapeDtypeStruct((2 * NP, D), jnp.float32),
    mesh=_mesh,
    scratch_types=[
        pltpu.VMEM((EP,), jnp.int32),
        pltpu.VMEM((EP,), jnp.int32),
        pltpu.VMEM((CH, D), jnp.float32),
        pltpu.VMEM((CH, D), jnp.float32),
        pltpu.VMEM_SHARED((NP, D), jnp.float32),
        pltpu.SemaphoreType.DMA,
        pltpu.SemaphoreType.DMA,
    ],
)
def _sc_scatter(src_hbm, dst_hbm, g_hbm, zeros_hbm, out_hbm,
                idx_s, idx_d, rows0, rows1, acc, sem0, sem1):
    c = lax.axis_index("c")
    s = lax.axis_index("s")
    wid = s * NC + c
    pltpu.sync_copy(zeros_hbm.at[pl.ds(s * RT, RT)], acc.at[pl.ds(s * RT, RT)])
    pltpu.sync_copy(src_hbm.at[pl.ds(wid * EP, EP)], idx_s)
    pltpu.sync_copy(dst_hbm.at[pl.ds(wid * EP, EP)], idx_d)
    plsc.subcore_barrier()

    def gather(j, rows, sem):
        return pltpu.async_copy(g_hbm.at[idx_s.at[pl.ds(j * CH, CH)]],
                                rows, sem)

    def gather_wait(j, rows, sem):
        pltpu.make_async_copy(g_hbm.at[idx_s.at[pl.ds(j * CH, CH)]],
                              rows, sem).wait()

    def scatter(j, rows):
        pltpu.sync_copy(rows, acc.at[idx_d.at[pl.ds(j * CH, CH)]], add=True)

    gather(0, rows0, sem0)

    def body(t, carry):
        j0 = 2 * t
        gather(j0 + 1, rows1, sem1)
        gather_wait(j0, rows0, sem0)
        scatter(j0, rows0)
        gather(j0 + 2, rows0, sem0)
        gather_wait(j0 + 1, rows1, sem1)
        scatter(j0 + 1, rows1)
        return carry

    lax.fori_loop(0, (NCH - 1) // 2, body, 0)
    gather_wait(NCH - 1, rows0, sem0)
    scatter(NCH - 1, rows0)

    plsc.subcore_barrier()
    pltpu.sync_copy(acc.at[pl.ds(s * RT, RT)],
                    out_hbm.at[pl.ds(c * NP + s * RT, RT)])


BLK = 2048
NBLK = NP // BLK


def _tc_first_body(deg2_ref, x_ref, w_ref, g_ref):
    dinv = lax.rsqrt(deg2_ref[0] + deg2_ref[1] + 1.0)
    g_ref[...] = jnp.dot(x_ref[...], w_ref[...],
                         preferred_element_type=jnp.float32) * dinv


def _tc_mid_body(deg2_ref, a_ref, g_ref, b_ref, w_ref, out_ref):
    dinv = lax.rsqrt(deg2_ref[0] + deg2_ref[1] + 1.0)
    h = (a_ref[0] + a_ref[1] + g_ref[...]) * dinv + b_ref[...]
    h = jnp.maximum(h, 0.0)
    out_ref[...] = jnp.dot(h, w_ref[...],
                           preferred_element_type=jnp.float32) * dinv


def _tc_pool_body(deg2_ref, a_ref, g_ref, b_ref, batch_ref, out_ref, cnt_ref):
    i = pl.program_id(0)
    dinv = lax.rsqrt(deg2_ref[0] + deg2_ref[1] + 1.0)
    h = (a_ref[0] + a_ref[1] + g_ref[...]) * dinv + b_ref[...]
    bt = batch_ref[...]
    gid = lax.broadcasted_iota(jnp.int32, (G, BLK), 0)
    onehot = (gid == bt).astype(jnp.float32)

    @pl.when(i == 0)
    def _():
        out_ref[...] = jnp.zeros_like(out_ref)
        cnt_ref[...] = jnp.zeros_like(cnt_ref)

    out_ref[...] += jnp.dot(onehot, h, preferred_element_type=jnp.float32)
    cnt_ref[...] += jnp.sum(onehot, axis=1, keepdims=True)

    @pl.when(i == NBLK - 1)
    def _():
        out_ref[...] = out_ref[...] / jnp.maximum(cnt_ref[...], 1.0)


def _tc_first(deg2, x, w):
    return pl.pallas_call(
        _tc_first_body,
        grid=(NBLK,),
        in_specs=[
            pl.BlockSpec((2, BLK, 1), lambda i: (0, i, 0)),
            pl.BlockSpec((BLK, D), lambda i: (i, 0)),
            pl.BlockSpec((D, D), lambda i: (0, 0)),
        ],
        out_specs=pl.BlockSpec((BLK, D), lambda i: (i, 0)),
        out_shape=jax.ShapeDtypeStruct((NP, D), jnp.float32),
    )(deg2, x, w)


def _tc_mid(deg2, a, g, b, w):
    return pl.pallas_call(
        _tc_mid_body,
        grid=(NBLK,),
        in_specs=[
            pl.BlockSpec((2, BLK, 1), lambda i: (0, i, 0)),
            pl.BlockSpec((2, BLK, D), lambda i: (0, i, 0)),
            pl.BlockSpec((BLK, D), lambda i: (i, 0)),
            pl.BlockSpec((1, D), lambda i: (0, 0)),
            pl.BlockSpec((D, D), lambda i: (0, 0)),
        ],
        out_specs=pl.BlockSpec((BLK, D), lambda i: (i, 0)),
        out_shape=jax.ShapeDtypeStruct((NP, D), jnp.float32),
    )(deg2, a, g, b, w)


def _tc_pool(deg2, a, g, b, batch_row):
    out, _ = pl.pallas_call(
        _tc_pool_body,
        grid=(NBLK,),
        in_specs=[
            pl.BlockSpec((2, BLK, 1), lambda i: (0, i, 0)),
            pl.BlockSpec((2, BLK, D), lambda i: (0, i, 0)),
            pl.BlockSpec((BLK, D), lambda i: (i, 0)),
            pl.BlockSpec((1, D), lambda i: (0, 0)),
            pl.BlockSpec((1, BLK), lambda i: (0, i)),
        ],
        out_specs=[
            pl.BlockSpec((G, D), lambda i: (0, 0)),
            pl.BlockSpec((G, 1), lambda i: (0, 0)),
        ],
        out_shape=[
            jax.ShapeDtypeStruct((G, D), jnp.float32),
            jax.ShapeDtypeStruct((G, 1), jnp.float32),
        ],
    )(deg2, a, g, b, batch_row)
    return out


def kernel(x, edge_index, batch, W1, b1, W2, b2, W3, b3):
    x = x.astype(jnp.float32)
    src = edge_index[0]
    dst = edge_index[1]

    ones_rows = jnp.zeros((CH, DW), jnp.float32).at[:, 0].set(1.0)
    zeros_deg = jnp.zeros((NP, DW), jnp.float32)
    zeros_nd = jnp.zeros((NP, D), jnp.float32)

    deg_out = _sc_degree(dst, ones_rows, zeros_deg)
    deg2 = deg_out.reshape(2, NP, DW)[:, :, 0:1]

    b1r = b1.reshape(1, D)
    b2r = b2.reshape(1, D)
    b3r = b3.reshape(1, D)
    batch_row = jnp.pad(batch, (0, NP - N), constant_values=-1).reshape(1, NP)
    xp = jnp.pad(x, ((0, NP - N), (0, 0)))

    g1 = _tc_first(deg2, xp, W1)
    a1 = _sc_scatter(src, dst, g1, zeros_nd).reshape(2, NP, D)
    g2 = _tc_mid(deg2, a1, g1, b1r, W2)
    a2 = _sc_scatter(src, dst, g2, zeros_nd).reshape(2, NP, D)
    g3 = _tc_mid(deg2, a2, g2, b2r, W3)
    a3 = _sc_scatter(src, dst, g3, zeros_nd).reshape(2, NP, D)
    return _tc_pool(deg2, a3, g3, b3r, batch_row)


# speedup vs baseline: 26.9072x; 1.1082x over previous
import functools

import jax
import jax.numpy as jnp
from jax import lax
from jax.experimental import pallas as pl
from jax.experimental.pallas import tpu as pltpu
from jax.experimental.pallas import tpu_sc as plsc

N = 10000
E = 320000
D = 128
G = 128

NC = 2
NS = 16
NW = NC * NS
EP = E // NW
CH = 40
NCH = EP // CH
RT = 640
NP = 10240
DW = 128

_mesh = plsc.VectorSubcoreMesh(core_axis_name="c", subcore_axis_name="s")


@functools.partial(
    pl.kernel,
    out_type=jax.ShapeDtypeStruct((2 * NP, DW), jnp.float32),
    mesh=_mesh,
    scratch_types=[
        pltpu.VMEM((EP,), jnp.int32),
        pltpu.VMEM((CH, DW), jnp.float32),
        pltpu.VMEM_SHARED((NP, DW), jnp.float32),
        pltpu.SemaphoreType.DMA,
    ],
)
def _sc_degree(dst_hbm, ones_hbm, zeros_hbm, out_hbm, idx_d, ones_v, acc, sem):
    c = lax.axis_index("c")
    s = lax.axis_index("s")
    wid = s * NC + c
    stripe = NP // NS
    pltpu.sync_copy(zeros_hbm.at[pl.ds(s * stripe, stripe)],
                    acc.at[pl.ds(s * stripe, stripe)])
    pltpu.sync_copy(ones_hbm, ones_v)
    pltpu.sync_copy(dst_hbm.at[pl.ds(wid * EP, EP)], idx_d)
    plsc.subcore_barrier()

    def body(j, carry):
        pltpu.sync_copy(ones_v, acc.at[idx_d.at[pl.ds(j * CH, CH)]], add=True)
        return carry

    lax.fori_loop(0, NCH, body, 0)
    plsc.subcore_barrier()
    pltpu.sync_copy(acc.at[pl.ds(s * stripe, stripe)],
                    out_hbm.at[pl.ds(c * NP + s * stripe, stripe)])


@functools.partial(
    pl.kernel,
    out_type=jax.ShapeDtypeStruct((2 * NP, D), jnp.float32),
    mesh=_mesh,
    scratch_types=[
        pltpu.VMEM((EP,), jnp.int32),
        pltpu.VMEM((EP,), jnp.int32),
        pltpu.VMEM((CH, D), jnp.float32),
        pltpu.VMEM((CH, D), jnp.float32),
        pltpu.VMEM((CH, D), jnp.float32),
        pltpu.VMEM((CH, D), jnp.float32),
        pltpu.VMEM_SHARED((NP, D), jnp.float32),
        pltpu.SemaphoreType.DMA,
        pltpu.SemaphoreType.DMA,
        pltpu.SemaphoreType.DMA,
        pltpu.SemaphoreType.DMA,
    ],
)
def _sc_scatter(src_hbm, dst_hbm, g_hbm, zeros_hbm, out_hbm,
                idx_s, idx_d, rows0, rows1, rows2, rows3, acc,
                sem0, sem1, sem2, sem3):
    c = lax.axis_index("c")
    s = lax.axis_index("s")
    wid = s * NC + c
    pltpu.sync_copy(zeros_hbm.at[pl.ds(s * RT, RT)], acc.at[pl.ds(s * RT, RT)])
    pltpu.sync_copy(src_hbm.at[pl.ds(wid * EP, EP)], idx_s)
    pltpu.sync_copy(dst_hbm.at[pl.ds(wid * EP, EP)], idx_d)
    plsc.subcore_barrier()

    def gather(j, rows, sem):
        return pltpu.async_copy(g_hbm.at[idx_s.at[pl.ds(j * CH, CH)]],
                                rows, sem)

    def gather_wait(j, rows, sem):
        pltpu.make_async_copy(g_hbm.at[idx_s.at[pl.ds(j * CH, CH)]],
                              rows, sem).wait()

    def scatter(j, rows):
        pltpu.sync_copy(rows, acc.at[idx_d.at[pl.ds(j * CH, CH)]], add=True)

    bufs = (rows0, rows1, rows2, rows3)
    sems = (sem0, sem1, sem2, sem3)
    for k in range(4):
        gather(k, bufs[k], sems[k])

    def body(t, carry):
        j0 = 4 * t
        for k in range(4):
            gather_wait(j0 + k, bufs[k], sems[k])
            scatter(j0 + k, bufs[k])
            gather(j0 + k + 4, bufs[k], sems[k])
        return carry

    nb = NCH // 4 - 1
    lax.fori_loop(0, nb, body, 0)
    for j in range(4 * nb, NCH):
        k = j % 4
        gather_wait(j, bufs[k], sems[k])
        scatter(j, bufs[k])
        if j + 4 < NCH:
            gather(j + 4, bufs[k], sems[k])

    plsc.subcore_barrier()
    pltpu.sync_copy(acc.at[pl.ds(s * RT, RT)],
                    out_hbm.at[pl.ds(c * NP + s * RT, RT)])


BLK = 2048
NBLK = NP // BLK


def _tc_first_body(deg2_ref, x_ref, w_ref, g_ref):
    dinv = lax.rsqrt(deg2_ref[0] + deg2_ref[1] + 1.0)
    g_ref[...] = jnp.dot(x_ref[...], w_ref[...],
                         preferred_element_type=jnp.float32) * dinv


def _tc_mid_body(deg2_ref, a_ref, g_ref, b_ref, w_ref, out_ref):
    dinv = lax.rsqrt(deg2_ref[0] + deg2_ref[1] + 1.0)
    h = (a_ref[0] + a_ref[1] + g_ref[...]) * dinv + b_ref[...]
    h = jnp.maximum(h, 0.0)
    out_ref[...] = jnp.dot(h, w_ref[...],
                           preferred_element_type=jnp.float32) * dinv


def _tc_pool_body(deg2_ref, a_ref, g_ref, b_ref, batch_ref, out_ref, cnt_ref):
    i = pl.program_id(0)
    dinv = lax.rsqrt(deg2_ref[0] + deg2_ref[1] + 1.0)
    h = (a_ref[0] + a_ref[1] + g_ref[...]) * dinv + b_ref[...]
    bt = batch_ref[...]
    gid = lax.broadcasted_iota(jnp.int32, (G, BLK), 0)
    onehot = (gid == bt).astype(jnp.float32)

    @pl.when(i == 0)
    def _():
        out_ref[...] = jnp.zeros_like(out_ref)
        cnt_ref[...] = jnp.zeros_like(cnt_ref)

    out_ref[...] += jnp.dot(onehot, h, preferred_element_type=jnp.float32)
    cnt_ref[...] += jnp.sum(onehot, axis=1, keepdims=True)

    @pl.when(i == NBLK - 1)
    def _():
        out_ref[...] = out_ref[...] / jnp.maximum(cnt_ref[...], 1.0)


def _tc_first(deg2, x, w):
    return pl.pallas_call(
        _tc_first_body,
        grid=(NBLK,),
        in_specs=[
            pl.BlockSpec((2, BLK, 1), lambda i: (0, i, 0)),
            pl.BlockSpec((BLK, D), lambda i: (i, 0)),
            pl.BlockSpec((D, D), lambda i: (0, 0)),
        ],
        out_specs=pl.BlockSpec((BLK, D), lambda i: (i, 0)),
        out_shape=jax.ShapeDtypeStruct((NP, D), jnp.float32),
    )(deg2, x, w)


def _tc_mid(deg2, a, g, b, w):
    return pl.pallas_call(
        _tc_mid_body,
        grid=(NBLK,),
        in_specs=[
            pl.BlockSpec((2, BLK, 1), lambda i: (0, i, 0)),
            pl.BlockSpec((2, BLK, D), lambda i: (0, i, 0)),
            pl.BlockSpec((BLK, D), lambda i: (i, 0)),
            pl.BlockSpec((1, D), lambda i: (0, 0)),
            pl.BlockSpec((D, D), lambda i: (0, 0)),
        ],
        out_specs=pl.BlockSpec((BLK, D), lambda i: (i, 0)),
        out_shape=jax.ShapeDtypeStruct((NP, D), jnp.float32),
    )(deg2, a, g, b, w)


def _tc_pool(deg2, a, g, b, batch_row):
    out, _ = pl.pallas_call(
        _tc_pool_body,
        grid=(NBLK,),
        in_specs=[
            pl.BlockSpec((2, BLK, 1), lambda i: (0, i, 0)),
            pl.BlockSpec((2, BLK, D), lambda i: (0, i, 0)),
            pl.BlockSpec((BLK, D), lambda i: (i, 0)),
            pl.BlockSpec((1, D), lambda i: (0, 0)),
            pl.BlockSpec((1, BLK), lambda i: (0, i)),
        ],
        out_specs=[
            pl.BlockSpec((G, D), lambda i: (0, 0)),
            pl.BlockSpec((G, 1), lambda i: (0, 0)),
        ],
        out_shape=[
            jax.ShapeDtypeStruct((G, D), jnp.float32),
            jax.ShapeDtypeStruct((G, 1), jnp.float32),
        ],
    )(deg2, a, g, b, batch_row)
    return out


def kernel(x, edge_index, batch, W1, b1, W2, b2, W3, b3):
    x = x.astype(jnp.float32)
    src = edge_index[0]
    dst = edge_index[1]

    ones_rows = jnp.zeros((CH, DW), jnp.float32).at[:, 0].set(1.0)
    zeros_deg = jnp.zeros((NP, DW), jnp.float32)
    zeros_nd = jnp.zeros((NP, D), jnp.float32)

    deg_out = _sc_degree(dst, ones_rows, zeros_deg)
    deg2 = deg_out.reshape(2, NP, DW)[:, :, 0:1]

    b1r = b1.reshape(1, D)
    b2r = b2.reshape(1, D)
    b3r = b3.reshape(1, D)
    batch_row = jnp.pad(batch, (0, NP - N), constant_values=-1).reshape(1, NP)
    xp = jnp.pad(x, ((0, NP - N), (0, 0)))

    g1 = _tc_first(deg2, xp, W1)
    a1 = _sc_scatter(src, dst, g1, zeros_nd).reshape(2, NP, D)
    g2 = _tc_mid(deg2, a1, g1, b1r, W2)
    a2 = _sc_scatter(src, dst, g2, zeros_nd).reshape(2, NP, D)
    g3 = _tc_mid(deg2, a2, g2, b2r, W3)
    a3 = _sc_scatter(src, dst, g3, zeros_nd).reshape(2, NP, D)
    return _tc_pool(deg2, a3, g3, b3r, batch_row)


# speedup vs baseline: 28.5149x; 1.0597x over previous
import functools

import jax
import jax.numpy as jnp
from jax import lax
from jax.experimental import pallas as pl
from jax.experimental.pallas import tpu as pltpu
from jax.experimental.pallas import tpu_sc as plsc

N = 10000
E = 320000
D = 128
G = 128

NC = 2
NS = 16
NW = NC * NS
EP = E // NW
CH = 40
NCH = EP // CH
DCH = 80
NDCH = EP // DCH
RT = 640
NP = 10240
DW = 128

_mesh = plsc.VectorSubcoreMesh(core_axis_name="c", subcore_axis_name="s")


@functools.partial(
    pl.kernel,
    out_type=jax.ShapeDtypeStruct((2 * NP, DW), jnp.float32),
    mesh=_mesh,
    scratch_types=[
        pltpu.VMEM((EP,), jnp.int32),
        pltpu.VMEM((DCH, DW), jnp.float32),
        pltpu.VMEM_SHARED((NP, DW), jnp.float32),
        pltpu.SemaphoreType.DMA,
    ],
)
def _sc_degree(dst_hbm, ones_hbm, zeros_hbm, out_hbm, idx_d, ones_v, acc, sem):
    c = lax.axis_index("c")
    s = lax.axis_index("s")
    wid = s * NC + c
    stripe = NP // NS
    pltpu.sync_copy(zeros_hbm.at[pl.ds(s * stripe, stripe)],
                    acc.at[pl.ds(s * stripe, stripe)])
    pltpu.sync_copy(ones_hbm, ones_v)
    pltpu.sync_copy(dst_hbm.at[pl.ds(wid * EP, EP)], idx_d)
    plsc.subcore_barrier()

    def body(j, carry):
        pltpu.sync_copy(ones_v, acc.at[idx_d.at[pl.ds(j * DCH, DCH)]],
                        add=True)
        return carry

    lax.fori_loop(0, NDCH, body, 0)
    plsc.subcore_barrier()
    pltpu.sync_copy(acc.at[pl.ds(s * stripe, stripe)],
                    out_hbm.at[pl.ds(c * NP + s * stripe, stripe)])


@functools.partial(
    pl.kernel,
    out_type=jax.ShapeDtypeStruct((2 * NP, D), jnp.float32),
    mesh=_mesh,
    scratch_types=[
        pltpu.VMEM((EP,), jnp.int32),
        pltpu.VMEM((EP,), jnp.int32),
        pltpu.VMEM((CH, D), jnp.float32),
        pltpu.VMEM((CH, D), jnp.float32),
        pltpu.VMEM((CH, D), jnp.float32),
        pltpu.VMEM((CH, D), jnp.float32),
        pltpu.VMEM((CH, D), jnp.float32),
        pltpu.VMEM_SHARED((NP, D), jnp.float32),
        pltpu.SemaphoreType.DMA,
        pltpu.SemaphoreType.DMA,
        pltpu.SemaphoreType.DMA,
        pltpu.SemaphoreType.DMA,
        pltpu.SemaphoreType.DMA,
    ],
)
def _sc_scatter(src_hbm, dst_hbm, g_hbm, zeros_hbm, out_hbm,
                idx_s, idx_d, rows0, rows1, rows2, rows3, rows4, acc,
                sem0, sem1, sem2, sem3, sem4):
    c = lax.axis_index("c")
    s = lax.axis_index("s")
    wid = s * NC + c
    pltpu.sync_copy(zeros_hbm.at[pl.ds(s * RT, RT)], acc.at[pl.ds(s * RT, RT)])
    pltpu.sync_copy(src_hbm.at[pl.ds(wid * EP, EP)], idx_s)
    pltpu.sync_copy(dst_hbm.at[pl.ds(wid * EP, EP)], idx_d)
    plsc.subcore_barrier()

    def gather(j, rows, sem):
        return pltpu.async_copy(g_hbm.at[idx_s.at[pl.ds(j * CH, CH)]],
                                rows, sem)

    def gather_wait(j, rows, sem):
        pltpu.make_async_copy(g_hbm.at[idx_s.at[pl.ds(j * CH, CH)]],
                              rows, sem).wait()

    def scatter(j, rows):
        pltpu.sync_copy(rows, acc.at[idx_d.at[pl.ds(j * CH, CH)]], add=True)

    bufs = (rows0, rows1, rows2, rows3, rows4)
    sems = (sem0, sem1, sem2, sem3, sem4)
    P = len(bufs)
    for k in range(P):
        gather(k, bufs[k], sems[k])

    def body(t, carry):
        j0 = P * t
        for k in range(P):
            gather_wait(j0 + k, bufs[k], sems[k])
            scatter(j0 + k, bufs[k])
            gather(j0 + k + P, bufs[k], sems[k])
        return carry

    nb = NCH // P - 1
    lax.fori_loop(0, nb, body, 0)
    for j in range(P * nb, NCH):
        k = j % P
        gather_wait(j, bufs[k], sems[k])
        scatter(j, bufs[k])
        if j + P < NCH:
            gather(j + P, bufs[k], sems[k])

    plsc.subcore_barrier()
    pltpu.sync_copy(acc.at[pl.ds(s * RT, RT)],
                    out_hbm.at[pl.ds(c * NP + s * RT, RT)])


BLK = 2048
NBLK = NP // BLK


def _tc_first_body(deg2_ref, x_ref, w_ref, g_ref):
    dinv = lax.rsqrt(deg2_ref[0] + deg2_ref[1] + 1.0)
    g_ref[...] = jnp.dot(x_ref[...], w_ref[...],
                         preferred_element_type=jnp.float32) * dinv


def _tc_mid_body(deg2_ref, a_ref, g_ref, b_ref, w_ref, out_ref):
    dinv = lax.rsqrt(deg2_ref[0] + deg2_ref[1] + 1.0)
    h = (a_ref[0] + a_ref[1] + g_ref[...]) * dinv + b_ref[...]
    h = jnp.maximum(h, 0.0)
    out_ref[...] = jnp.dot(h, w_ref[...],
                           preferred_element_type=jnp.float32) * dinv


def _tc_pool_body(deg2_ref, a_ref, g_ref, b_ref, batch_ref, out_ref, cnt_ref):
    i = pl.program_id(0)
    dinv = lax.rsqrt(deg2_ref[0] + deg2_ref[1] + 1.0)
    h = (a_ref[0] + a_ref[1] + g_ref[...]) * dinv + b_ref[...]
    bt = batch_ref[...]
    gid = lax.broadcasted_iota(jnp.int32, (G, BLK), 0)
    onehot = (gid == bt).astype(jnp.float32)

    @pl.when(i == 0)
    def _():
        out_ref[...] = jnp.zeros_like(out_ref)
        cnt_ref[...] = jnp.zeros_like(cnt_ref)

    out_ref[...] += jnp.dot(onehot, h, preferred_element_type=jnp.float32)
    cnt_ref[...] += jnp.sum(onehot, axis=1, keepdims=True)

    @pl.when(i == NBLK - 1)
    def _():
        out_ref[...] = out_ref[...] / jnp.maximum(cnt_ref[...], 1.0)


def _tc_first(deg2, x, w):
    return pl.pallas_call(
        _tc_first_body,
        grid=(NBLK,),
        in_specs=[
            pl.BlockSpec((2, BLK, 1), lambda i: (0, i, 0)),
            pl.BlockSpec((BLK, D), lambda i: (i, 0)),
            pl.BlockSpec((D, D), lambda i: (0, 0)),
        ],
        out_specs=pl.BlockSpec((BLK, D), lambda i: (i, 0)),
        out_shape=jax.ShapeDtypeStruct((NP, D), jnp.float32),
    )(deg2, x, w)


def _tc_mid(deg2, a, g, b, w):
    return pl.pallas_call(
        _tc_mid_body,
        grid=(NBLK,),
        in_specs=[
            pl.BlockSpec((2, BLK, 1), lambda i: (0, i, 0)),
            pl.BlockSpec((2, BLK, D), lambda i: (0, i, 0)),
            pl.BlockSpec((BLK, D), lambda i: (i, 0)),
            pl.BlockSpec((1, D), lambda i: (0, 0)),
            pl.BlockSpec((D, D), lambda i: (0, 0)),
        ],
        out_specs=pl.BlockSpec((BLK, D), lambda i: (i, 0)),
        out_shape=jax.ShapeDtypeStruct((NP, D), jnp.float32),
    )(deg2, a, g, b, w)


def _tc_pool(deg2, a, g, b, batch_row):
    out, _ = pl.pallas_call(
        _tc_pool_body,
        grid=(NBLK,),
        in_specs=[
            pl.BlockSpec((2, BLK, 1), lambda i: (0, i, 0)),
            pl.BlockSpec((2, BLK, D), lambda i: (0, i, 0)),
            pl.BlockSpec((BLK, D), lambda i: (i, 0)),
            pl.BlockSpec((1, D), lambda i: (0, 0)),
            pl.BlockSpec((1, BLK), lambda i: (0, i)),
        ],
        out_specs=[
            pl.BlockSpec((G, D), lambda i: (0, 0)),
            pl.BlockSpec((G, 1), lambda i: (0, 0)),
        ],
        out_shape=[
            jax.ShapeDtypeStruct((G, D), jnp.float32),
            jax.ShapeDtypeStruct((G, 1), jnp.float32),
        ],
    )(deg2, a, g, b, batch_row)
    return out


def kernel(x, edge_index, batch, W1, b1, W2, b2, W3, b3):
    x = x.astype(jnp.float32)
    src = edge_index[0]
    dst = edge_index[1]

    ones_rows = jnp.zeros((DCH, DW), jnp.float32).at[:, 0].set(1.0)
    zeros_deg = jnp.zeros((NP, DW), jnp.float32)
    zeros_nd = jnp.zeros((NP, D), jnp.float32)

    deg_out = _sc_degree(dst, ones_rows, zeros_deg)
    deg2 = deg_out.reshape(2, NP, DW)[:, :, 0:1]

    b1r = b1.reshape(1, D)
    b2r = b2.reshape(1, D)
    b3r = b3.reshape(1, D)
    batch_row = jnp.pad(batch, (0, NP - N), constant_values=-1).reshape(1, NP)
    xp = jnp.pad(x, ((0, NP - N), (0, 0)))

    g1 = _tc_first(deg2, xp, W1)
    a1 = _sc_scatter(src, dst, g1, zeros_nd).reshape(2, NP, D)
    g2 = _tc_mid(deg2, a1, g1, b1r, W2)
    a2 = _sc_scatter(src, dst, g2, zeros_nd).reshape(2, NP, D)
    g3 = _tc_mid(deg2, a2, g2, b2r, W3)
    a3 = _sc_scatter(src, dst, g3, zeros_nd).reshape(2, NP, D)
    return _tc_pool(deg2, a3, g3, b3r, batch_row)


# speedup vs baseline: 28.6785x; 1.0057x over previous
import functools

import jax
import jax.numpy as jnp
from jax import lax
from jax.experimental import pallas as pl
from jax.experimental.pallas import tpu as pltpu
from jax.experimental.pallas import tpu_sc as plsc

N = 10000
E = 320000
D = 128
G = 128

NC = 2
NS = 16
NW = NC * NS
EP = E // NW
CH = 40
NCH = EP // CH
DCH = 80
NDCH = EP // DCH
RT = 640
NP = 10240
DW = 128

_mesh = plsc.VectorSubcoreMesh(core_axis_name="c", subcore_axis_name="s")


@functools.partial(
    pl.kernel,
    out_type=jax.ShapeDtypeStruct((2 * NP, DW), jnp.float32),
    mesh=_mesh,
    scratch_types=[
        pltpu.VMEM((EP,), jnp.int32),
        pltpu.VMEM((DCH, DW), jnp.float32),
        pltpu.VMEM_SHARED((NP, DW), jnp.float32),
        pltpu.SemaphoreType.DMA,
    ],
)
def _sc_degree(dst_hbm, ones_hbm, zeros_hbm, out_hbm, idx_d, ones_v, acc, sem):
    c = lax.axis_index("c")
    s = lax.axis_index("s")
    wid = s * NC + c
    stripe = NP // NS
    pltpu.sync_copy(zeros_hbm.at[pl.ds(s * stripe, stripe)],
                    acc.at[pl.ds(s * stripe, stripe)])
    pltpu.sync_copy(ones_hbm, ones_v)
    pltpu.sync_copy(dst_hbm.at[pl.ds(wid * EP, EP)], idx_d)
    plsc.subcore_barrier()

    def body(j, carry):
        pltpu.sync_copy(ones_v, acc.at[idx_d.at[pl.ds(j * DCH, DCH)]],
                        add=True)
        return carry

    lax.fori_loop(0, NDCH, body, 0)
    plsc.subcore_barrier()
    pltpu.sync_copy(acc.at[pl.ds(s * stripe, stripe)],
                    out_hbm.at[pl.ds(c * NP + s * stripe, stripe)])


@functools.partial(
    pl.kernel,
    out_type=jax.ShapeDtypeStruct((2 * NP, D), jnp.float32),
    mesh=_mesh,
    scratch_types=[
        pltpu.VMEM((EP,), jnp.int32),
        pltpu.VMEM((7 * CH,), jnp.int32),
        pltpu.VMEM((CH, D), jnp.float32),
        pltpu.VMEM((CH, D), jnp.float32),
        pltpu.VMEM((CH, D), jnp.float32),
        pltpu.VMEM((CH, D), jnp.float32),
        pltpu.VMEM((CH, D), jnp.float32),
        pltpu.VMEM((CH, D), jnp.float32),
        pltpu.VMEM((CH, D), jnp.float32),
        pltpu.VMEM_SHARED((NP, D), jnp.float32),
        pltpu.SemaphoreType.DMA,
        pltpu.SemaphoreType.DMA,
        pltpu.SemaphoreType.DMA,
        pltpu.SemaphoreType.DMA,
        pltpu.SemaphoreType.DMA,
        pltpu.SemaphoreType.DMA,
        pltpu.SemaphoreType.DMA,
    ],
)
def _sc_scatter(src_hbm, dst_hbm, g_hbm, zeros_hbm, out_hbm,
                idx_s, idx_d, rows0, rows1, rows2, rows3, rows4, rows5, rows6,
                acc, sem0, sem1, sem2, sem3, sem4, sem5, sem6):
    c = lax.axis_index("c")
    s = lax.axis_index("s")
    wid = s * NC + c
    pltpu.sync_copy(zeros_hbm.at[pl.ds(s * RT, RT)], acc.at[pl.ds(s * RT, RT)])
    pltpu.sync_copy(src_hbm.at[pl.ds(wid * EP, EP)], idx_s)
    plsc.subcore_barrier()

    bufs = (rows0, rows1, rows2, rows3, rows4, rows5, rows6)
    sems = (sem0, sem1, sem2, sem3, sem4, sem5, sem6)
    P = len(bufs)

    def gather(j, k):
        pltpu.async_copy(g_hbm.at[idx_s.at[pl.ds(j * CH, CH)]],
                         bufs[k], sems[k])
        pltpu.async_copy(dst_hbm.at[pl.ds(wid * EP + j * CH, CH)],
                         idx_d.at[pl.ds(k * CH, CH)], sems[k])

    def drain(j, k):
        pltpu.make_async_copy(g_hbm.at[idx_s.at[pl.ds(j * CH, CH)]],
                              bufs[k], sems[k]).wait()
        pltpu.make_async_copy(dst_hbm.at[pl.ds(wid * EP + j * CH, CH)],
                              idx_d.at[pl.ds(k * CH, CH)], sems[k]).wait()
        pltpu.sync_copy(bufs[k], acc.at[idx_d.at[pl.ds(k * CH, CH)]],
                        add=True)

    for k in range(P):
        gather(k, k)

    def body(t, carry):
        j0 = P * t
        for k in range(P):
            drain(j0 + k, k)
            gather(j0 + k + P, k)
        return carry

    nb = NCH // P - 1
    lax.fori_loop(0, nb, body, 0)
    for j in range(P * nb, NCH):
        k = j % P
        drain(j, k)
        if j + P < NCH:
            gather(j + P, k)

    plsc.subcore_barrier()
    pltpu.sync_copy(acc.at[pl.ds(s * RT, RT)],
                    out_hbm.at[pl.ds(c * NP + s * RT, RT)])


BLK = 2048
NBLK = NP // BLK


def _tc_first_body(deg2_ref, x_ref, w_ref, g_ref):
    dinv = lax.rsqrt(deg2_ref[0] + deg2_ref[1] + 1.0)
    g_ref[...] = jnp.dot(x_ref[...], w_ref[...],
                         preferred_element_type=jnp.float32) * dinv


def _tc_mid_body(deg2_ref, a_ref, g_ref, b_ref, w_ref, out_ref):
    dinv = lax.rsqrt(deg2_ref[0] + deg2_ref[1] + 1.0)
    h = (a_ref[0] + a_ref[1] + g_ref[...]) * dinv + b_ref[...]
    h = jnp.maximum(h, 0.0)
    out_ref[...] = jnp.dot(h, w_ref[...],
                           preferred_element_type=jnp.float32) * dinv


def _tc_pool_body(deg2_ref, a_ref, g_ref, b_ref, batch_ref, out_ref, cnt_ref):
    i = pl.program_id(0)
    dinv = lax.rsqrt(deg2_ref[0] + deg2_ref[1] + 1.0)
    h = (a_ref[0] + a_ref[1] + g_ref[...]) * dinv + b_ref[...]
    bt = batch_ref[...]
    gid = lax.broadcasted_iota(jnp.int32, (G, BLK), 0)
    onehot = (gid == bt).astype(jnp.float32)

    @pl.when(i == 0)
    def _():
        out_ref[...] = jnp.zeros_like(out_ref)
        cnt_ref[...] = jnp.zeros_like(cnt_ref)

    out_ref[...] += jnp.dot(onehot, h, preferred_element_type=jnp.float32)
    cnt_ref[...] += jnp.sum(onehot, axis=1, keepdims=True)

    @pl.when(i == NBLK - 1)
    def _():
        out_ref[...] = out_ref[...] / jnp.maximum(cnt_ref[...], 1.0)


def _tc_first(deg2, x, w):
    return pl.pallas_call(
        _tc_first_body,
        grid=(NBLK,),
        in_specs=[
            pl.BlockSpec((2, BLK, 1), lambda i: (0, i, 0)),
            pl.BlockSpec((BLK, D), lambda i: (i, 0)),
            pl.BlockSpec((D, D), lambda i: (0, 0)),
        ],
        out_specs=pl.BlockSpec((BLK, D), lambda i: (i, 0)),
        out_shape=jax.ShapeDtypeStruct((NP, D), jnp.float32),
    )(deg2, x, w)


def _tc_mid(deg2, a, g, b, w):
    return pl.pallas_call(
        _tc_mid_body,
        grid=(NBLK,),
        in_specs=[
            pl.BlockSpec((2, BLK, 1), lambda i: (0, i, 0)),
            pl.BlockSpec((2, BLK, D), lambda i: (0, i, 0)),
            pl.BlockSpec((BLK, D), lambda i: (i, 0)),
            pl.BlockSpec((1, D), lambda i: (0, 0)),
            pl.BlockSpec((D, D), lambda i: (0, 0)),
        ],
        out_specs=pl.BlockSpec((BLK, D), lambda i: (i, 0)),
        out_shape=jax.ShapeDtypeStruct((NP, D), jnp.float32),
    )(deg2, a, g, b, w)


def _tc_pool(deg2, a, g, b, batch_row):
    out, _ = pl.pallas_call(
        _tc_pool_body,
        grid=(NBLK,),
        in_specs=[
            pl.BlockSpec((2, BLK, 1), lambda i: (0, i, 0)),
            pl.BlockSpec((2, BLK, D), lambda i: (0, i, 0)),
            pl.BlockSpec((BLK, D), lambda i: (i, 0)),
            pl.BlockSpec((1, D), lambda i: (0, 0)),
            pl.BlockSpec((1, BLK), lambda i: (0, i)),
        ],
        out_specs=[
            pl.BlockSpec((G, D), lambda i: (0, 0)),
            pl.BlockSpec((G, 1), lambda i: (0, 0)),
        ],
        out_shape=[
            jax.ShapeDtypeStruct((G, D), jnp.float32),
            jax.ShapeDtypeStruct((G, 1), jnp.float32),
        ],
    )(deg2, a, g, b, batch_row)
    return out


def kernel(x, edge_index, batch, W1, b1, W2, b2, W3, b3):
    x = x.astype(jnp.float32)
    src = edge_index[0]
    dst = edge_index[1]

    ones_rows = jnp.zeros((DCH, DW), jnp.float32).at[:, 0].set(1.0)
    zeros_deg = jnp.zeros((NP, DW), jnp.float32)
    zeros_nd = jnp.zeros((NP, D), jnp.float32)

    deg_out = _sc_degree(dst, ones_rows, zeros_deg)
    deg2 = deg_out.reshape(2, NP, DW)[:, :, 0:1]

    b1r = b1.reshape(1, D)
    b2r = b2.reshape(1, D)
    b3r = b3.reshape(1, D)
    batch_row = jnp.pad(batch, (0, NP - N), constant_values=-1).reshape(1, NP)
    xp = jnp.pad(x, ((0, NP - N), (0, 0)))

    g1 = _tc_first(deg2, xp, W1)
    a1 = _sc_scatter(src, dst, g1, zeros_nd).reshape(2, NP, D)
    g2 = _tc_mid(deg2, a1, g1, b1r, W2)
    a2 = _sc_scatter(src, dst, g2, zeros_nd).reshape(2, NP, D)
    g3 = _tc_mid(deg2, a2, g2, b2r, W3)
    a3 = _sc_scatter(src, dst, g3, zeros_nd).reshape(2, NP, D)
    return _tc_pool(deg2, a3, g3, b3r, batch_row)


# speedup vs baseline: 28.7463x; 1.0024x over previous
import functools

import jax
import jax.numpy as jnp
from jax import lax
from jax.experimental import pallas as pl
from jax.experimental.pallas import tpu as pltpu
from jax.experimental.pallas import tpu_sc as plsc

N = 10000
E = 320000
D = 128
G = 128

NC = 2
NS = 16
NW = NC * NS
EP = E // NW
CH = 40
NCH = EP // CH
DCH = 80
NDCH = EP // DCH
RT = 640
NP = 10240
DW = 128

_mesh = plsc.VectorSubcoreMesh(core_axis_name="c", subcore_axis_name="s")


@functools.partial(
    pl.kernel,
    out_type=jax.ShapeDtypeStruct((2 * NP, DW), jnp.float32),
    mesh=_mesh,
    scratch_types=[
        pltpu.VMEM((EP,), jnp.int32),
        pltpu.VMEM((DCH, DW), jnp.float32),
        pltpu.VMEM_SHARED((NP, DW), jnp.float32),
        pltpu.SemaphoreType.DMA,
    ],
)
def _sc_degree(dst_hbm, ones_hbm, zeros_hbm, out_hbm, idx_d, ones_v, acc, sem):
    c = lax.axis_index("c")
    s = lax.axis_index("s")
    wid = s * NC + c
    stripe = NP // NS
    pltpu.sync_copy(zeros_hbm, ones_v)
    for r in range(stripe // DCH):
        pltpu.sync_copy(ones_v, acc.at[pl.ds(s * stripe + r * DCH, DCH)])
    pltpu.sync_copy(ones_hbm, ones_v)
    pltpu.sync_copy(dst_hbm.at[pl.ds(wid * EP, EP)], idx_d)
    plsc.subcore_barrier()

    def body(j, carry):
        pltpu.sync_copy(ones_v, acc.at[idx_d.at[pl.ds(j * DCH, DCH)]],
                        add=True)
        return carry

    lax.fori_loop(0, NDCH, body, 0)
    plsc.subcore_barrier()
    pltpu.sync_copy(acc.at[pl.ds(s * stripe, stripe)],
                    out_hbm.at[pl.ds(c * NP + s * stripe, stripe)])


@functools.partial(
    pl.kernel,
    out_type=jax.ShapeDtypeStruct((2 * NP, D), jnp.float32),
    mesh=_mesh,
    scratch_types=[
        pltpu.VMEM((EP,), jnp.int32),
        pltpu.VMEM((7 * CH,), jnp.int32),
        pltpu.VMEM((CH, D), jnp.float32),
        pltpu.VMEM((CH, D), jnp.float32),
        pltpu.VMEM((CH, D), jnp.float32),
        pltpu.VMEM((CH, D), jnp.float32),
        pltpu.VMEM((CH, D), jnp.float32),
        pltpu.VMEM((CH, D), jnp.float32),
        pltpu.VMEM((CH, D), jnp.float32),
        pltpu.VMEM_SHARED((NP, D), jnp.float32),
        pltpu.SemaphoreType.DMA,
        pltpu.SemaphoreType.DMA,
        pltpu.SemaphoreType.DMA,
        pltpu.SemaphoreType.DMA,
        pltpu.SemaphoreType.DMA,
        pltpu.SemaphoreType.DMA,
        pltpu.SemaphoreType.DMA,
    ],
)
def _sc_scatter(src_hbm, dst_hbm, g_hbm, zeros_hbm, out_hbm,
                idx_s, idx_d, rows0, rows1, rows2, rows3, rows4, rows5, rows6,
                acc, sem0, sem1, sem2, sem3, sem4, sem5, sem6):
    c = lax.axis_index("c")
    s = lax.axis_index("s")
    wid = s * NC + c
    pltpu.sync_copy(zeros_hbm, rows0)
    for r in range(RT // CH):
        pltpu.sync_copy(rows0, acc.at[pl.ds(s * RT + r * CH, CH)])
    pltpu.sync_copy(src_hbm.at[pl.ds(wid * EP, EP)], idx_s)
    plsc.subcore_barrier()

    bufs = (rows0, rows1, rows2, rows3, rows4, rows5, rows6)
    sems = (sem0, sem1, sem2, sem3, sem4, sem5, sem6)
    P = len(bufs)

    def gather(j, k):
        pltpu.async_copy(g_hbm.at[idx_s.at[pl.ds(j * CH, CH)]],
                         bufs[k], sems[k])
        pltpu.async_copy(dst_hbm.at[pl.ds(wid * EP + j * CH, CH)],
                         idx_d.at[pl.ds(k * CH, CH)], sems[k])

    def drain(j, k):
        pltpu.make_async_copy(g_hbm.at[idx_s.at[pl.ds(j * CH, CH)]],
                              bufs[k], sems[k]).wait()
        pltpu.make_async_copy(dst_hbm.at[pl.ds(wid * EP + j * CH, CH)],
                              idx_d.at[pl.ds(k * CH, CH)], sems[k]).wait()
        pltpu.sync_copy(bufs[k], acc.at[idx_d.at[pl.ds(k * CH, CH)]],
                        add=True)

    for k in range(P):
        gather(k, k)

    def body(t, carry):
        j0 = P * t
        for k in range(P):
            drain(j0 + k, k)
            gather(j0 + k + P, k)
        return carry

    nb = NCH // P - 1
    lax.fori_loop(0, nb, body, 0)
    for j in range(P * nb, NCH):
        k = j % P
        drain(j, k)
        if j + P < NCH:
            gather(j + P, k)

    plsc.subcore_barrier()
    pltpu.sync_copy(acc.at[pl.ds(s * RT, RT)],
                    out_hbm.at[pl.ds(c * NP + s * RT, RT)])


BLK = 2048
NBLK = NP // BLK


def _tc_first_body(deg2_ref, x_ref, w_ref, g_ref):
    dinv = lax.rsqrt(deg2_ref[0] + deg2_ref[1] + 1.0)
    g_ref[...] = jnp.dot(x_ref[...], w_ref[...],
                         preferred_element_type=jnp.float32) * dinv


def _tc_mid_body(deg2_ref, a_ref, g_ref, b_ref, w_ref, out_ref):
    dinv = lax.rsqrt(deg2_ref[0] + deg2_ref[1] + 1.0)
    h = (a_ref[0] + a_ref[1] + g_ref[...]) * dinv + b_ref[...]
    h = jnp.maximum(h, 0.0)
    out_ref[...] = jnp.dot(h, w_ref[...],
                           preferred_element_type=jnp.float32) * dinv


def _tc_pool_body(deg2_ref, a_ref, g_ref, b_ref, batch_ref, out_ref, cnt_ref):
    i = pl.program_id(0)
    dinv = lax.rsqrt(deg2_ref[0] + deg2_ref[1] + 1.0)
    h = (a_ref[0] + a_ref[1] + g_ref[...]) * dinv + b_ref[...]
    bt = batch_ref[...]
    gid = lax.broadcasted_iota(jnp.int32, (G, BLK), 0)
    onehot = (gid == bt).astype(jnp.float32)

    @pl.when(i == 0)
    def _():
        out_ref[...] = jnp.zeros_like(out_ref)
        cnt_ref[...] = jnp.zeros_like(cnt_ref)

    out_ref[...] += jnp.dot(onehot, h, preferred_element_type=jnp.float32)
    cnt_ref[...] += jnp.sum(onehot, axis=1, keepdims=True)

    @pl.when(i == NBLK - 1)
    def _():
        out_ref[...] = out_ref[...] / jnp.maximum(cnt_ref[...], 1.0)


def _tc_first(deg2, x, w):
    return pl.pallas_call(
        _tc_first_body,
        grid=(NBLK,),
        in_specs=[
            pl.BlockSpec((2, BLK, 1), lambda i: (0, i, 0)),
            pl.BlockSpec((BLK, D), lambda i: (i, 0)),
            pl.BlockSpec((D, D), lambda i: (0, 0)),
        ],
        out_specs=pl.BlockSpec((BLK, D), lambda i: (i, 0)),
        out_shape=jax.ShapeDtypeStruct((NP, D), jnp.float32),
    )(deg2, x, w)


def _tc_mid(deg2, a, g, b, w):
    return pl.pallas_call(
        _tc_mid_body,
        grid=(NBLK,),
        in_specs=[
            pl.BlockSpec((2, BLK, 1), lambda i: (0, i, 0)),
            pl.BlockSpec((2, BLK, D), lambda i: (0, i, 0)),
            pl.BlockSpec((BLK, D), lambda i: (i, 0)),
            pl.BlockSpec((1, D), lambda i: (0, 0)),
            pl.BlockSpec((D, D), lambda i: (0, 0)),
        ],
        out_specs=pl.BlockSpec((BLK, D), lambda i: (i, 0)),
        out_shape=jax.ShapeDtypeStruct((NP, D), jnp.float32),
    )(deg2, a, g, b, w)


def _tc_pool(deg2, a, g, b, batch_row):
    out, _ = pl.pallas_call(
        _tc_pool_body,
        grid=(NBLK,),
        in_specs=[
            pl.BlockSpec((2, BLK, 1), lambda i: (0, i, 0)),
            pl.BlockSpec((2, BLK, D), lambda i: (0, i, 0)),
            pl.BlockSpec((BLK, D), lambda i: (i, 0)),
            pl.BlockSpec((1, D), lambda i: (0, 0)),
            pl.BlockSpec((1, BLK), lambda i: (0, i)),
        ],
        out_specs=[
            pl.BlockSpec((G, D), lambda i: (0, 0)),
            pl.BlockSpec((G, 1), lambda i: (0, 0)),
        ],
        out_shape=[
            jax.ShapeDtypeStruct((G, D), jnp.float32),
            jax.ShapeDtypeStruct((G, 1), jnp.float32),
        ],
    )(deg2, a, g, b, batch_row)
    return out


def kernel(x, edge_index, batch, W1, b1, W2, b2, W3, b3):
    x = x.astype(jnp.float32)
    src = edge_index[0]
    dst = edge_index[1]

    ones_rows = jnp.zeros((DCH, DW), jnp.float32).at[:, 0].set(1.0)
    zeros_deg = jnp.zeros((DCH, DW), jnp.float32)
    zeros_nd = jnp.zeros((CH, D), jnp.float32)

    deg_out = _sc_degree(dst, ones_rows, zeros_deg)
    deg2 = deg_out.reshape(2, NP, DW)[:, :, 0:1]

    b1r = b1.reshape(1, D)
    b2r = b2.reshape(1, D)
    b3r = b3.reshape(1, D)
    batch_row = jnp.pad(batch, (0, NP - N), constant_values=-1).reshape(1, NP)
    xp = jnp.pad(x, ((0, NP - N), (0, 0)))

    g1 = _tc_first(deg2, xp, W1)
    a1 = _sc_scatter(src, dst, g1, zeros_nd).reshape(2, NP, D)
    g2 = _tc_mid(deg2, a1, g1, b1r, W2)
    a2 = _sc_scatter(src, dst, g2, zeros_nd).reshape(2, NP, D)
    g3 = _tc_mid(deg2, a2, g2, b2r, W3)
    a3 = _sc_scatter(src, dst, g3, zeros_nd).reshape(2, NP, D)
    return _tc_pool(deg2, a3, g3, b3r, batch_row)
